# baseline (speedup 1.0000x reference)
import jax
import jax.numpy as jnp
from jax import lax
from jax.experimental import pallas as pl
from jax.experimental.pallas import tpu as pltpu
from jax.experimental.pallas import tpu_sc as plsc

_B = 4
_S = 4096
_C = 2048
_W = 16
_RPW = 512
_WPB = 8


def _sc_body(x_hbm, out_hbm, buf, obuf):
    cid = lax.axis_index("c")
    sid = lax.axis_index("s")
    wid = sid * 2 + cid
    b = wid // _WPB
    r0 = (wid % _WPB) * _RPW

    def chunk(c, carry):
        row = r0 + c * _W
        g = b * _S + row
        halo = b * _S + (row + _S - 1) % _S
        pltpu.sync_copy(x_hbm.at[pl.ds(g * _C, _W * _C)],
                        buf.at[pl.ds(_C, _W * _C)])
        pltpu.sync_copy(x_hbm.at[pl.ds(halo * _C, _C)],
                        buf.at[pl.ds(0, _C)])

        def grp(j, carry2):
            o = j * 16
            obuf[pl.ds(o, 16)] = buf[pl.ds(o + _C, 16)] - buf[pl.ds(o, 16)]
            return carry2

        lax.fori_loop(0, _W * _C // 16, grp, 0)
        pltpu.sync_copy(obuf, out_hbm.at[pl.ds(g * _C, _W * _C)])
        return carry

    lax.fori_loop(0, _RPW // _W, chunk, 0)


def kernel(x):
    B, S, C = x.shape
    xf = x.reshape(-1)
    mesh = plsc.VectorSubcoreMesh(core_axis_name="c", subcore_axis_name="s")
    out = pl.kernel(
        _sc_body,
        out_type=jax.ShapeDtypeStruct((B * S * C,), x.dtype),
        scratch_types=[
            pltpu.VMEM(((_W + 1) * _C,), jnp.float32),
            pltpu.VMEM((_W * _C,), jnp.float32),
        ],
        mesh=mesh,
    )(xf)
    return out.reshape(B, S, C)


# speedup vs baseline: 2.0816x; 2.0816x over previous
import jax
import jax.numpy as jnp
from jax import lax
from jax.experimental import pallas as pl
from jax.experimental.pallas import tpu as pltpu
from jax.experimental.pallas import tpu_sc as plsc

_B = 4
_S = 4096
_C = 2048
_W = 16
_NCHUNK = 32
_RPW = _W * _NCHUNK
_WPB = 8
_INW = (_W + 1) * _C
_OUTW = _W * _C


def _compute(buf, obuf):
    @plsc.parallel_loop(0, _C // 16)
    def _(gi):
        o = gi * 16
        prev = buf[pl.ds(o, 16)]
        for r in range(_W):
            cur = buf[pl.ds(o + (r + 1) * _C, 16)]
            obuf[pl.ds(o + r * _C, 16)] = cur - prev
            prev = cur


def _sc_body(x_hbm, out_hbm, buf0, buf1, obuf, sem0, sem1, semo):
    cid = lax.axis_index("c")
    sid = lax.axis_index("s")
    wid = sid * 2 + cid
    b = wid // _WPB
    r0 = (wid % _WPB) * _RPW
    g0 = b * _S + r0

    bufs = (buf0, buf1)
    sems = (sem0, sem1)

    def start_in(c, buf, sem):
        g = g0 + c * _W
        pltpu.make_async_copy(
            x_hbm.at[pl.ds((g - 1) * _C, _INW)], buf, sem).start()

    def wait_in(c, buf, sem):
        pltpu.make_async_copy(
            x_hbm.at[pl.ds(0, _INW)], buf, sem).wait()

    def start_out(c):
        g = g0 + c * _W
        pltpu.make_async_copy(
            obuf, out_hbm.at[pl.ds(g * _C, _OUTW)], semo).start()

    def wait_out():
        pltpu.make_async_copy(
            obuf, out_hbm.at[pl.ds(g0 * _C, _OUTW)], semo).wait()

    halo0 = b * _S + (r0 + _S - 1) % _S
    pltpu.make_async_copy(
        x_hbm.at[pl.ds(halo0 * _C, _C)], buf0.at[pl.ds(0, _C)], sem0).start()
    pltpu.make_async_copy(
        x_hbm.at[pl.ds(g0 * _C, _W * _C)], buf0.at[pl.ds(_C, _W * _C)],
        sem0).start()
    start_in(1, buf1, sem1)

    def pair(k, carry):
        c0 = 2 * k
        c1 = c0 + 1
        wait_in(c0, bufs[0], sems[0])

        @pl.when(k > 0)
        def _():
            wait_out()
        _compute(bufs[0], obuf)
        start_out(c0)

        @pl.when(c0 + 2 < _NCHUNK)
        def _():
            start_in(c0 + 2, bufs[0], sems[0])

        wait_in(c1, bufs[1], sems[1])
        wait_out()
        _compute(bufs[1], obuf)
        start_out(c1)

        @pl.when(c1 + 2 < _NCHUNK)
        def _():
            start_in(c1 + 2, bufs[1], sems[1])

        return carry

    lax.fori_loop(0, _NCHUNK // 2, pair, 0)
    wait_out()


def kernel(x):
    B, S, C = x.shape
    xf = x.reshape(-1)
    mesh = plsc.VectorSubcoreMesh(core_axis_name="c", subcore_axis_name="s")
    out = pl.kernel(
        _sc_body,
        out_type=jax.ShapeDtypeStruct((B * S * C,), x.dtype),
        scratch_types=[
            pltpu.VMEM((_INW,), jnp.float32),
            pltpu.VMEM((_INW,), jnp.float32),
            pltpu.VMEM((_OUTW,), jnp.float32),
            pltpu.SemaphoreType.DMA,
            pltpu.SemaphoreType.DMA,
            pltpu.SemaphoreType.DMA,
        ],
        mesh=mesh,
    )(xf)
    return out.reshape(B, S, C)


# speedup vs baseline: 5.8074x; 2.7898x over previous
import jax
import jax.numpy as jnp
from jax import lax
from jax.experimental import pallas as pl
from jax.experimental.pallas import tpu as pltpu
from jax.experimental.pallas import tpu_sc as plsc

_B = 4
_S = 4096
_C = 2048
_W = 16
_NCHUNK = 32
_RPW = _W * _NCHUNK
_WPB = 8
_HROW = 7


def _compute(buf, hbuf, obuf):
    @plsc.parallel_loop(0, _C // 16)
    def _(gi):
        o = gi * 16
        prev = hbuf[_HROW, pl.ds(o, 16)]
        for r in range(_W):
            cur = buf[r, pl.ds(o, 16)]
            obuf[r, pl.ds(o, 16)] = cur - prev
            prev = cur


def _save_halo(buf, hbuf):
    @plsc.parallel_loop(0, _C // 16)
    def _(gi):
        o = gi * 16
        hbuf[_HROW, pl.ds(o, 16)] = buf[_W - 1, pl.ds(o, 16)]


def _sc_body(x_hbm, out_hbm, buf0, buf1, obuf, hbuf, sem0, sem1, semo):
    cid = lax.axis_index("c")
    sid = lax.axis_index("s")
    wid = sid * 2 + cid
    b = wid // _WPB
    r0 = (wid % _WPB) * _RPW
    g0 = b * _S + r0

    bufs = (buf0, buf1)
    sems = (sem0, sem1)

    def start_in(c, buf, sem):
        g = pl.multiple_of(g0 + c * _W, 8)
        pltpu.make_async_copy(x_hbm.at[pl.ds(g, _W)], buf, sem).start()

    def wait_in(buf, sem):
        pltpu.make_async_copy(x_hbm.at[pl.ds(0, _W)], buf, sem).wait()

    def start_out(c):
        g = pl.multiple_of(g0 + c * _W, 8)
        pltpu.make_async_copy(obuf, out_hbm.at[pl.ds(g, _W)], semo).start()

    def wait_out():
        pltpu.make_async_copy(obuf, out_hbm.at[pl.ds(g0, _W)], semo).wait()

    halo_hi = b * _S + (r0 + _S - 1) % _S + 1
    halo_lo = pl.multiple_of(halo_hi - 8, 8)
    pltpu.make_async_copy(x_hbm.at[pl.ds(halo_lo, 8)], hbuf, sem0).start()
    start_in(0, buf0, sem0)
    start_in(1, buf1, sem1)
    pltpu.make_async_copy(x_hbm.at[pl.ds(0, 8)], hbuf, sem0).wait()

    def step(c, buf, sem, is_first, is_last2):
        wait_in(buf, sem)
        if not is_first:
            wait_out()
        _compute(buf, hbuf, obuf)
        _save_halo(buf, hbuf)
        start_out(c)
        if not is_last2:
            @pl.when(c + 2 < _NCHUNK)
            def _():
                start_in(c + 2, buf, sem)

    def pair(k, carry):
        c0 = 2 * k
        step(c0, bufs[0], sems[0], False, False)
        step(c0 + 1, bufs[1], sems[1], False, False)
        return carry

    step(0, bufs[0], sems[0], True, False)
    step(1, bufs[1], sems[1], False, False)
    lax.fori_loop(1, _NCHUNK // 2, pair, 0)
    wait_out()


def kernel(x):
    B, S, C = x.shape
    x2 = x.reshape(B * S, C)
    mesh = plsc.VectorSubcoreMesh(core_axis_name="c", subcore_axis_name="s")
    out = pl.kernel(
        _sc_body,
        out_type=jax.ShapeDtypeStruct((B * S, C), x.dtype),
        scratch_types=[
            pltpu.VMEM((_W, _C), jnp.float32),
            pltpu.VMEM((_W, _C), jnp.float32),
            pltpu.VMEM((_W, _C), jnp.float32),
            pltpu.VMEM((8, _C), jnp.float32),
            pltpu.SemaphoreType.DMA,
            pltpu.SemaphoreType.DMA,
            pltpu.SemaphoreType.DMA,
        ],
        mesh=mesh,
    )(x2)
    return out.reshape(B, S, C)


# speedup vs baseline: 5.9227x; 1.0199x over previous
import jax
import jax.numpy as jnp
from jax import lax
from jax.experimental import pallas as pl
from jax.experimental.pallas import tpu as pltpu
from jax.experimental.pallas import tpu_sc as plsc

_B = 4
_S = 4096
_C = 2048
_W = 8
_NCHUNK = 64
_RPW = _W * _NCHUNK
_WPB = 8
_HROW = 7


def _compute(buf, hbuf, obuf):
    @plsc.parallel_loop(0, _C // 16)
    def _(gi):
        o = gi * 16
        prev = hbuf[_HROW, pl.ds(o, 16)]
        for r in range(_W):
            cur = buf[r, pl.ds(o, 16)]
            obuf[r, pl.ds(o, 16)] = cur - prev
            prev = cur


def _save_halo(buf, hbuf):
    @plsc.parallel_loop(0, _C // 16)
    def _(gi):
        o = gi * 16
        hbuf[_HROW, pl.ds(o, 16)] = buf[_W - 1, pl.ds(o, 16)]


def _sc_body(x_hbm, out_hbm, buf0, buf1, obuf0, obuf1, hbuf,
             sem0, sem1, semo0, semo1):
    cid = lax.axis_index("c")
    sid = lax.axis_index("s")
    wid = sid * 2 + cid
    b = wid // _WPB
    r0 = (wid % _WPB) * _RPW
    g0 = b * _S + r0

    bufs = (buf0, buf1)
    sems = (sem0, sem1)
    obufs = (obuf0, obuf1)
    osems = (semo0, semo1)

    def start_in(c, buf, sem):
        g = pl.multiple_of(g0 + c * _W, 8)
        pltpu.make_async_copy(x_hbm.at[pl.ds(g, _W)], buf, sem).start()

    def wait_in(buf, sem):
        pltpu.make_async_copy(x_hbm.at[pl.ds(0, _W)], buf, sem).wait()

    def start_out(c, obuf, osem):
        g = pl.multiple_of(g0 + c * _W, 8)
        pltpu.make_async_copy(obuf, out_hbm.at[pl.ds(g, _W)], osem).start()

    def wait_out(obuf, osem):
        pltpu.make_async_copy(obuf, out_hbm.at[pl.ds(g0, _W)], osem).wait()

    halo_hi = b * _S + (r0 + _S - 1) % _S + 1
    halo_lo = pl.multiple_of(halo_hi - 8, 8)
    pltpu.make_async_copy(x_hbm.at[pl.ds(halo_lo, 8)], hbuf, sem0).start()
    start_in(0, buf0, sem0)
    start_in(1, buf1, sem1)
    pltpu.make_async_copy(x_hbm.at[pl.ds(0, 8)], hbuf, sem0).wait()

    def step(c, p, is_first):
        wait_in(bufs[p], sems[p])
        if not is_first:
            wait_out(obufs[p], osems[p])
        _compute(bufs[p], hbuf, obufs[p])
        _save_halo(bufs[p], hbuf)
        start_out(c, obufs[p], osems[p])

        @pl.when(c + 2 < _NCHUNK)
        def _():
            start_in(c + 2, bufs[p], sems[p])

    def pair(k, carry):
        c0 = 2 * k
        step(c0, 0, False)
        step(c0 + 1, 1, False)
        return carry

    step(0, 0, True)
    step(1, 1, True)
    lax.fori_loop(1, _NCHUNK // 2, pair, 0)
    wait_out(obufs[0], osems[0])
    wait_out(obufs[1], osems[1])


def kernel(x):
    B, S, C = x.shape
    x2 = x.reshape(B * S, C)
    mesh = plsc.VectorSubcoreMesh(core_axis_name="c", subcore_axis_name="s")
    out = pl.kernel(
        _sc_body,
        out_type=jax.ShapeDtypeStruct((B * S, C), x.dtype),
        scratch_types=[
            pltpu.VMEM((_W, _C), jnp.float32),
            pltpu.VMEM((_W, _C), jnp.float32),
            pltpu.VMEM((_W, _C), jnp.float32),
            pltpu.VMEM((_W, _C), jnp.float32),
            pltpu.VMEM((8, _C), jnp.float32),
            pltpu.SemaphoreType.DMA,
            pltpu.SemaphoreType.DMA,
            pltpu.SemaphoreType.DMA,
            pltpu.SemaphoreType.DMA,
        ],
        mesh=mesh,
    )(x2)
    return out.reshape(B, S, C)


# speedup vs baseline: 6.3731x; 1.0760x over previous
import jax
import jax.numpy as jnp
from jax import lax
from jax.experimental import pallas as pl
from jax.experimental.pallas import tpu as pltpu
from jax.experimental.pallas import tpu_sc as plsc

_B = 4
_S = 4096
_C = 2048
_W = 8
_NCHUNK = 64
_RPW = _W * _NCHUNK
_WPB = 8
_NIN = 4
_HROW = 7


def _compute(buf, hbuf, obuf):
    @plsc.parallel_loop(0, _C // 16)
    def _(gi):
        o = gi * 16
        prev = hbuf[_HROW, pl.ds(o, 16)]
        for r in range(_W):
            cur = buf[r, pl.ds(o, 16)]
            obuf[r, pl.ds(o, 16)] = cur - prev
            prev = cur
        hbuf[_HROW, pl.ds(o, 16)] = prev


def _sc_body(x_hbm, out_hbm, buf0, buf1, buf2, buf3, obuf0, obuf1, hbuf,
             sem0, sem1, sem2, sem3, semo0, semo1, semh):
    cid = lax.axis_index("c")
    sid = lax.axis_index("s")
    wid = sid * 2 + cid
    b = wid // _WPB
    r0 = (wid % _WPB) * _RPW
    g0 = b * _S + r0

    bufs = (buf0, buf1, buf2, buf3)
    sems = (sem0, sem1, sem2, sem3)
    obufs = (obuf0, obuf1)
    osems = (semo0, semo1)

    def start_in(c, buf, sem):
        g = pl.multiple_of(g0 + c * _W, 8)
        pltpu.make_async_copy(x_hbm.at[pl.ds(g, _W)], buf, sem).start()

    def wait_in(buf, sem):
        pltpu.make_async_copy(x_hbm.at[pl.ds(0, _W)], buf, sem).wait()

    def start_out(c, obuf, osem):
        g = pl.multiple_of(g0 + c * _W, 8)
        pltpu.make_async_copy(obuf, out_hbm.at[pl.ds(g, _W)], osem).start()

    def wait_out(obuf, osem):
        pltpu.make_async_copy(obuf, out_hbm.at[pl.ds(g0, _W)], osem).wait()

    halo_hi = b * _S + (r0 + _S - 1) % _S + 1
    halo_lo = pl.multiple_of(halo_hi - 8, 8)
    pltpu.make_async_copy(x_hbm.at[pl.ds(halo_lo, 8)], hbuf, semh).start()
    for j in range(_NIN):
        start_in(j, bufs[j], sems[j])
    pltpu.make_async_copy(x_hbm.at[pl.ds(0, 8)], hbuf, semh).wait()

    def step(c, j, is_first):
        wait_in(bufs[j], sems[j])
        if not is_first:
            wait_out(obufs[j % 2], osems[j % 2])
        _compute(bufs[j], hbuf, obufs[j % 2])
        start_out(c, obufs[j % 2], osems[j % 2])

        @pl.when(c + _NIN < _NCHUNK)
        def _():
            start_in(c + _NIN, bufs[j], sems[j])

    def quad(k, carry):
        c0 = _NIN * k
        for j in range(_NIN):
            step(c0 + j, j, False)
        return carry

    for j in range(_NIN):
        step(j, j, j < 2)
    lax.fori_loop(1, _NCHUNK // _NIN, quad, 0)
    wait_out(obufs[0], osems[0])
    wait_out(obufs[1], osems[1])


def kernel(x):
    B, S, C = x.shape
    x2 = x.reshape(B * S, C)
    mesh = plsc.VectorSubcoreMesh(core_axis_name="c", subcore_axis_name="s")
    out = pl.kernel(
        _sc_body,
        out_type=jax.ShapeDtypeStruct((B * S, C), x.dtype),
        scratch_types=[
            pltpu.VMEM((_W, _C), jnp.float32),
            pltpu.VMEM((_W, _C), jnp.float32),
            pltpu.VMEM((_W, _C), jnp.float32),
            pltpu.VMEM((_W, _C), jnp.float32),
            pltpu.VMEM((_W, _C), jnp.float32),
            pltpu.VMEM((_W, _C), jnp.float32),
            pltpu.VMEM((8, _C), jnp.float32),
            pltpu.SemaphoreType.DMA,
            pltpu.SemaphoreType.DMA,
            pltpu.SemaphoreType.DMA,
            pltpu.SemaphoreType.DMA,
            pltpu.SemaphoreType.DMA,
            pltpu.SemaphoreType.DMA,
            pltpu.SemaphoreType.DMA,
        ],
        mesh=mesh,
    )(x2)
    return out.reshape(B, S, C)
